# baseline (speedup 1.0000x reference)
import functools

import jax
import jax.numpy as jnp
from jax import lax
from jax.experimental import pallas as pl
from jax.experimental.pallas import tpu as pltpu
from jax.experimental.pallas import tpu_sc as plsc

BATCH = 16384
MAXLEN = 50
VOCAB = 1000000
EMBED = 64
LANES = 16
NUM_WORKERS = 32

ROWS_PER_BLOCK = LANES
TOK_PER_BLOCK = ROWS_PER_BLOCK * MAXLEN
GATHER_CHUNKS = 8
CHUNK = TOK_PER_BLOCK // GATHER_CHUNKS
BLOCKS_TOTAL = BATCH // ROWS_PER_BLOCK
BLOCKS_PER_W = BLOCKS_TOTAL // NUM_WORKERS

TC_ROWS = 32768
TC_GRID = pl.cdiv(VOCAB, TC_ROWS)
PRE_LEN = TC_GRID * TC_ROWS


def _tc_pre_body(table_ref, w_ref, n_ref, d_ref):
    x = table_ref[...]
    w = w_ref[...]
    n_ref[...] = jnp.sum(x * x, axis=0)[None, None, :]
    d_ref[...] = jnp.sum(x * w, axis=0)[None, None, :]


_tc_pre = pl.pallas_call(
    _tc_pre_body,
    grid=(TC_GRID,),
    in_specs=[
        pl.BlockSpec((EMBED, TC_ROWS), lambda i: (0, i)),
        pl.BlockSpec((EMBED, 1), lambda i: (0, 0)),
    ],
    out_specs=[
        pl.BlockSpec((1, 1, TC_ROWS), lambda i: (i, 0, 0)),
        pl.BlockSpec((1, 1, TC_ROWS), lambda i: (i, 0, 0)),
    ],
    out_shape=[
        jax.ShapeDtypeStruct((TC_GRID, 1, TC_ROWS), jnp.float32),
        jax.ShapeDtypeStruct((TC_GRID, 1, TC_ROWS), jnp.float32),
    ],
)


def _sc_body(x_hbm, lens_hbm, nd_hbm, b_hbm, out_hbm,
             xslab_v, lens_slab_v, nd_v, b_v, out_slab_v,
             sem_g0, sem_g1, sem_s):
    wid = lax.axis_index("s") * 2 + lax.axis_index("c")
    base = wid * BLOCKS_PER_W
    c1 = pltpu.async_copy(x_hbm.at[pl.ds(base, BLOCKS_PER_W)], xslab_v, sem_s)
    c2 = pltpu.async_copy(
        lens_hbm.at[pl.ds(base * ROWS_PER_BLOCK,
                          BLOCKS_PER_W * ROWS_PER_BLOCK)], lens_slab_v, sem_s)
    c3 = pltpu.async_copy(b_hbm, b_v, sem_s)
    c1.wait(); c2.wait(); c3.wait()
    bias_vec = b_v[...]
    lane = lax.iota(jnp.int32, LANES)
    rowc = lane // 2
    colc = (lane % 2) * MAXLEN
    sems = (sem_g0, sem_g1)

    def fire(blk, p):
        for j in range(GATHER_CHUNKS):
            pltpu.async_copy(
                nd_hbm.at[xslab_v.at[blk, j]], nd_v.at[p, j], sems[p])

    def drain(blk, p):
        for j in range(GATHER_CHUNKS):
            pltpu.make_async_copy(
                nd_hbm.at[xslab_v.at[blk, j]], nd_v.at[p, j],
                sems[p]).wait()

    def compute(blk, p):
        lens_vec = lens_slab_v[pl.ds(blk * ROWS_PER_BLOCK, ROWS_PER_BLOCK)]
        num0 = jnp.zeros((LANES,), jnp.float32)
        den0 = jnp.zeros((LANES,), jnp.float32)
        num1 = jnp.zeros((LANES,), jnp.float32)
        den1 = jnp.zeros((LANES,), jnp.float32)
        zero16 = jnp.zeros((LANES,), jnp.int32)
        one16 = jnp.ones((LANES,), jnp.int32)
        for t in range(MAXLEN):
            col = colc + t
            nv = plsc.load_gather(nd_v.at[p], [rowc, col, zero16])
            dv = plsc.load_gather(nd_v.at[p], [rowc, col, one16])
            s = jnp.exp(nv)
            s = jnp.where(t < lens_vec, s, jnp.float32(0.0))
            if t % 2 == 0:
                num0 = num0 + s * dv
                den0 = den0 + s
            else:
                num1 = num1 + s * dv
                den1 = den1 + s
        score = (num0 + num1) / (den0 + den1) + bias_vec
        out_slab_v[pl.ds(blk * ROWS_PER_BLOCK, ROWS_PER_BLOCK)] = (
            1.0 / (1.0 + jnp.exp(-score)))

    fire(0, 0)

    @pl.loop(0, BLOCKS_PER_W // 2)
    def _(h):
        g0 = 2 * h
        g1 = g0 + 1
        fire(g1, 1)
        drain(g0, 0)
        compute(g0, 0)

        @pl.when(g0 + 2 < BLOCKS_PER_W)
        def _():
            fire(g0 + 2, 0)

        drain(g1, 1)
        compute(g1, 1)

    pltpu.sync_copy(
        out_slab_v,
        out_hbm.at[pl.ds(base * ROWS_PER_BLOCK,
                         BLOCKS_PER_W * ROWS_PER_BLOCK)])


_mesh = plsc.VectorSubcoreMesh(core_axis_name="c", subcore_axis_name="s")

_cp = pltpu.CompilerParams(
    needs_layout_passes=False, use_tc_tiling_on_sc=False)

_sc_pool = functools.partial(
    pl.kernel,
    compiler_params=_cp,
    out_type=jax.ShapeDtypeStruct((BATCH,), jnp.float32),
    mesh=_mesh,
    scratch_types=[
        pltpu.VMEM((BLOCKS_PER_W, GATHER_CHUNKS, CHUNK), jnp.int32),
        pltpu.VMEM((BLOCKS_PER_W * ROWS_PER_BLOCK,), jnp.int32),
        pltpu.VMEM((2, GATHER_CHUNKS, CHUNK, 2), jnp.float32),
        pltpu.VMEM((LANES,), jnp.float32),
        pltpu.VMEM((BLOCKS_PER_W * ROWS_PER_BLOCK,), jnp.float32),
        pltpu.SemaphoreType.DMA,
        pltpu.SemaphoreType.DMA,
        pltpu.SemaphoreType.DMA,
    ],
)(_sc_body)


def kernel(X, lens, table, W, b):
    assert X.shape == (BATCH, MAXLEN) and table.shape == (VOCAB, EMBED)
    norms2d, dots2d = _tc_pre(table.T, W.reshape(EMBED, 1))
    nd = jnp.stack([norms2d.reshape(PRE_LEN), dots2d.reshape(PRE_LEN)],
                   axis=1)
    x_blocks = X.reshape(BLOCKS_TOTAL, GATHER_CHUNKS, CHUNK)
    b16 = jnp.broadcast_to(b, (LANES,))
    prob = _sc_pool(x_blocks, lens, nd, b16)
    return prob.reshape(BATCH, 1)


# speedup vs baseline: 9.2251x; 9.2251x over previous
import functools

import jax
import jax.numpy as jnp
from jax import lax
from jax.experimental import pallas as pl
from jax.experimental.pallas import tpu as pltpu
from jax.experimental.pallas import tpu_sc as plsc

BATCH = 16384
MAXLEN = 50
VOCAB = 1000000
EMBED = 64
LANES = 16
NUM_WORKERS = 32

ROWS_PER_BLOCK = LANES
TOK_PER_BLOCK = ROWS_PER_BLOCK * MAXLEN
GATHER_CHUNKS = 8
CHUNK = TOK_PER_BLOCK // GATHER_CHUNKS
BLOCKS_TOTAL = BATCH // ROWS_PER_BLOCK
BLOCKS_PER_W = BLOCKS_TOTAL // NUM_WORKERS

TC_ROWS = 65536
TC_GRID = pl.cdiv(VOCAB, TC_ROWS)
PRE_LEN = TC_GRID * TC_ROWS


def _tc_pre_body(table_ref, w_ref, n_ref, d_ref):
    x = table_ref[...]
    w = w_ref[...]
    n_ref[...] = jnp.sum(x * x, axis=0)[None, None, :]
    d_ref[...] = jnp.sum(x * w, axis=0)[None, None, :]


_tc_pre = pl.pallas_call(
    _tc_pre_body,
    grid=(TC_GRID,),
    in_specs=[
        pl.BlockSpec((EMBED, TC_ROWS), lambda i: (0, i)),
        pl.BlockSpec((EMBED, 1), lambda i: (0, 0)),
    ],
    out_specs=[
        pl.BlockSpec((1, 1, TC_ROWS), lambda i: (i, 0, 0)),
        pl.BlockSpec((1, 1, TC_ROWS), lambda i: (i, 0, 0)),
    ],
    out_shape=[
        jax.ShapeDtypeStruct((TC_GRID, 1, TC_ROWS), jnp.float32),
        jax.ShapeDtypeStruct((TC_GRID, 1, TC_ROWS), jnp.float32),
    ],
)


def _sc_body(x_hbm, lens_hbm, norms_hbm, dots_hbm, b_hbm, out_hbm,
             xslab_v, lens_slab_v, nrm_v, dot_v, b_v, out_slab_v,
             sem_g0, sem_g1, sem_s):
    wid = lax.axis_index("s") * 2 + lax.axis_index("c")
    base = wid * BLOCKS_PER_W
    c1 = pltpu.async_copy(x_hbm.at[pl.ds(base, BLOCKS_PER_W)], xslab_v, sem_s)
    c2 = pltpu.async_copy(
        lens_hbm.at[pl.ds(base * ROWS_PER_BLOCK,
                          BLOCKS_PER_W * ROWS_PER_BLOCK)], lens_slab_v, sem_s)
    c3 = pltpu.async_copy(b_hbm, b_v, sem_s)
    c1.wait(); c2.wait(); c3.wait()
    bias_vec = b_v[...]
    lane = lax.iota(jnp.int32, LANES)
    rowc = lane // 2
    colc = (lane % 2) * MAXLEN
    sems = (sem_g0, sem_g1)

    def fire(blk, p):
        for j in range(GATHER_CHUNKS):
            pltpu.async_copy(
                norms_hbm.at[xslab_v.at[blk, j]], nrm_v.at[p, j], sems[p])
            pltpu.async_copy(
                dots_hbm.at[xslab_v.at[blk, j]], dot_v.at[p, j], sems[p])

    def drain(blk, p):
        for j in range(GATHER_CHUNKS):
            pltpu.make_async_copy(
                norms_hbm.at[xslab_v.at[blk, j]], nrm_v.at[p, j],
                sems[p]).wait()
            pltpu.make_async_copy(
                dots_hbm.at[xslab_v.at[blk, j]], dot_v.at[p, j],
                sems[p]).wait()

    def compute(blk, p):
        lens_vec = lens_slab_v[pl.ds(blk * ROWS_PER_BLOCK, ROWS_PER_BLOCK)]
        num0 = jnp.zeros((LANES,), jnp.float32)
        den0 = jnp.zeros((LANES,), jnp.float32)
        num1 = jnp.zeros((LANES,), jnp.float32)
        den1 = jnp.zeros((LANES,), jnp.float32)
        for t in range(MAXLEN):
            col = colc + t
            nv = plsc.load_gather(nrm_v.at[p], [rowc, col])
            dv = plsc.load_gather(dot_v.at[p], [rowc, col])
            s = jnp.exp(nv)
            s = jnp.where(t < lens_vec, s, jnp.float32(0.0))
            if t % 2 == 0:
                num0 = num0 + s * dv
                den0 = den0 + s
            else:
                num1 = num1 + s * dv
                den1 = den1 + s
        score = (num0 + num1) / (den0 + den1) + bias_vec
        out_slab_v[pl.ds(blk * ROWS_PER_BLOCK, ROWS_PER_BLOCK)] = (
            1.0 / (1.0 + jnp.exp(-score)))

    fire(0, 0)

    @pl.loop(0, BLOCKS_PER_W // 2)
    def _(h):
        g0 = 2 * h
        g1 = g0 + 1
        fire(g1, 1)
        drain(g0, 0)
        compute(g0, 0)

        @pl.when(g0 + 2 < BLOCKS_PER_W)
        def _():
            fire(g0 + 2, 0)

        drain(g1, 1)
        compute(g1, 1)

    pltpu.sync_copy(
        out_slab_v,
        out_hbm.at[pl.ds(base * ROWS_PER_BLOCK,
                         BLOCKS_PER_W * ROWS_PER_BLOCK)])


_mesh = plsc.VectorSubcoreMesh(core_axis_name="c", subcore_axis_name="s")

_cp = pltpu.CompilerParams(
    needs_layout_passes=False, use_tc_tiling_on_sc=False)

_sc_pool = functools.partial(
    pl.kernel,
    compiler_params=_cp,
    out_type=jax.ShapeDtypeStruct((BATCH,), jnp.float32),
    mesh=_mesh,
    scratch_types=[
        pltpu.VMEM((BLOCKS_PER_W, GATHER_CHUNKS, CHUNK), jnp.int32),
        pltpu.VMEM((BLOCKS_PER_W * ROWS_PER_BLOCK,), jnp.int32),
        pltpu.VMEM((2, GATHER_CHUNKS, CHUNK), jnp.float32),
        pltpu.VMEM((2, GATHER_CHUNKS, CHUNK), jnp.float32),
        pltpu.VMEM((LANES,), jnp.float32),
        pltpu.VMEM((BLOCKS_PER_W * ROWS_PER_BLOCK,), jnp.float32),
        pltpu.SemaphoreType.DMA,
        pltpu.SemaphoreType.DMA,
        pltpu.SemaphoreType.DMA,
    ],
)(_sc_body)


def kernel(X, lens, table, W, b):
    assert X.shape == (BATCH, MAXLEN) and table.shape == (VOCAB, EMBED)
    norms2d, dots2d = _tc_pre(table.T, W.reshape(EMBED, 1))
    norms = norms2d.reshape(PRE_LEN)
    dots = dots2d.reshape(PRE_LEN)
    x_blocks = X.reshape(BLOCKS_TOTAL, GATHER_CHUNKS, CHUNK)
    b16 = jnp.broadcast_to(b, (LANES,))
    prob = _sc_pool(x_blocks, lens, norms, dots, b16)
    return prob.reshape(BATCH, 1)
